# baseline (speedup 1.0000x reference)
import functools

import jax
import jax.numpy as jnp
from jax import lax
from jax.experimental import pallas as pl
from jax.experimental.pallas import tpu as pltpu
from jax.experimental.pallas import tpu_sc as plsc

D = 128
BATCH = 4
SEQ = 2048
N_TOK = BATCH * SEQ
NC = 2
NS = 16
NW = NC * NS
PER_W = N_TOK // NW
W_PER_ROW = SEQ // PER_W
SUB = 64
NSUB = PER_W // SUB

_mesh = plsc.VectorSubcoreMesh(core_axis_name="c", subcore_axis_name="s")


@functools.partial(
    pl.kernel,
    out_type=jax.ShapeDtypeStruct((N_TOK, D), jnp.float32),
    mesh=_mesh,
    scratch_types=[
        pltpu.VMEM((PER_W,), jnp.int32),
        pltpu.VMEM((PER_W,), jnp.int32),
        pltpu.VMEM((PER_W, D), jnp.float32),
        pltpu.SemaphoreType.DMA,
        pltpu.SemaphoreType.DMA,
        pltpu.SemaphoreType.DMA,
        pltpu.SemaphoreType.DMA,
        pltpu.SemaphoreType.DMA,
        pltpu.SemaphoreType.DMA,
        pltpu.SemaphoreType.DMA,
        pltpu.SemaphoreType.DMA,
        pltpu.SemaphoreType.DMA,
        pltpu.SemaphoreType.DMA,
        pltpu.SemaphoreType.DMA,
    ],
)
def _embed(ids_hbm, pos_hbm, wte_hbm, wpe_hbm, out_hbm,
           ti_v, pi_v, a,
           si0, si1, sa0, sa1, sa2, sa3, sb0, sb1, sb2, sb3, so):
    wid = lax.axis_index("s") * NC + lax.axis_index("c")
    brow = wid // W_PER_ROW
    s0 = (wid % W_PER_ROW) * PER_W
    ci0 = pltpu.async_copy(ids_hbm.at[brow, pl.ds(s0, PER_W)], ti_v, si0)
    ci1 = pltpu.async_copy(pos_hbm.at[brow, pl.ds(s0, PER_W)], pi_v, si1)
    sas = (sa0, sa1, sa2, sa3)
    sbs = (sb0, sb1, sb2, sb3)

    def _gather_wte(q):
        return pltpu.async_copy(
            wte_hbm.at[ti_v.at[pl.ds(q * SUB, SUB)]],
            a.at[pl.ds(q * SUB, SUB)], sas[q])

    ci0.wait()
    gas = [_gather_wte(0), _gather_wte(1)]
    ci1.wait()
    base = wid * PER_W
    gbs, cos = [], []
    for q in range(NSUB):
        gas[q].wait()
        gbs.append(pltpu.async_copy(
            wpe_hbm.at[pi_v.at[pl.ds(q * SUB, SUB)]],
            a.at[pl.ds(q * SUB, SUB)], sbs[q], add=True))
        if q + 2 < NSUB:
            gas.append(_gather_wte(q + 2))
        if q >= 1:
            gbs[q - 1].wait()
            cos.append(pltpu.async_copy(
                a.at[pl.ds((q - 1) * SUB, SUB)],
                out_hbm.at[pl.ds(base + (q - 1) * SUB, SUB)], so))
    gbs[NSUB - 1].wait()
    cos.append(pltpu.async_copy(
        a.at[pl.ds((NSUB - 1) * SUB, SUB)],
        out_hbm.at[pl.ds(base + (NSUB - 1) * SUB, SUB)], so))
    for co in cos:
        co.wait()


def kernel(input_ids, position_ids, wte, wpe):
    out = _embed(input_ids.astype(jnp.int32), position_ids.astype(jnp.int32),
                 wte, wpe)
    return out.reshape(input_ids.shape + (wte.shape[1],))


# speedup vs baseline: 1.0169x; 1.0169x over previous
import functools

import jax
import jax.numpy as jnp
from jax import lax
from jax.experimental import pallas as pl
from jax.experimental.pallas import tpu as pltpu
from jax.experimental.pallas import tpu_sc as plsc

D = 128
BATCH = 4
SEQ = 2048
N_TOK = BATCH * SEQ
NC = 2
NS = 16
NW = NC * NS
PER_W = N_TOK // NW
W_PER_ROW = SEQ // PER_W
SUB = 128
NSUB = PER_W // SUB

_mesh = plsc.VectorSubcoreMesh(core_axis_name="c", subcore_axis_name="s")


@functools.partial(
    pl.kernel,
    out_type=jax.ShapeDtypeStruct((N_TOK, D), jnp.float32),
    mesh=_mesh,
    scratch_types=[
        pltpu.VMEM((PER_W,), jnp.int32),
        pltpu.VMEM((PER_W,), jnp.int32),
        pltpu.VMEM((PER_W, D), jnp.float32),
        pltpu.SemaphoreType.DMA,
        pltpu.SemaphoreType.DMA,
        pltpu.SemaphoreType.DMA,
        pltpu.SemaphoreType.DMA,
        pltpu.SemaphoreType.DMA,
        pltpu.SemaphoreType.DMA,
        pltpu.SemaphoreType.DMA,
        pltpu.SemaphoreType.DMA,
        pltpu.SemaphoreType.DMA,
        pltpu.SemaphoreType.DMA,
        pltpu.SemaphoreType.DMA,
    ],
)
def _embed(ids_hbm, pos_hbm, wte_hbm, wpe_hbm, out_hbm,
           ti_v, pi_v, a,
           si0, si1, sa0, sa1, sa2, sa3, sb0, sb1, sb2, sb3, so):
    wid = lax.axis_index("s") * NC + lax.axis_index("c")
    brow = wid // W_PER_ROW
    s0 = (wid % W_PER_ROW) * PER_W
    ci0 = pltpu.async_copy(ids_hbm.at[brow, pl.ds(s0, PER_W)], ti_v, si0)
    ci1 = pltpu.async_copy(pos_hbm.at[brow, pl.ds(s0, PER_W)], pi_v, si1)
    sas = (sa0, sa1, sa2, sa3)
    sbs = (sb0, sb1, sb2, sb3)

    def _gather_wte(q):
        return pltpu.async_copy(
            wte_hbm.at[ti_v.at[pl.ds(q * SUB, SUB)]],
            a.at[pl.ds(q * SUB, SUB)], sas[q])

    ci0.wait()
    gas = [_gather_wte(0), _gather_wte(1)]
    ci1.wait()
    base = wid * PER_W
    gbs, cos = [], []
    for q in range(NSUB):
        gas[q].wait()
        gbs.append(pltpu.async_copy(
            wpe_hbm.at[pi_v.at[pl.ds(q * SUB, SUB)]],
            a.at[pl.ds(q * SUB, SUB)], sbs[q], add=True))
        if q + 2 < NSUB:
            gas.append(_gather_wte(q + 2))
        if q >= 1:
            gbs[q - 1].wait()
            cos.append(pltpu.async_copy(
                a.at[pl.ds((q - 1) * SUB, SUB)],
                out_hbm.at[pl.ds(base + (q - 1) * SUB, SUB)], so))
    gbs[NSUB - 1].wait()
    cos.append(pltpu.async_copy(
        a.at[pl.ds((NSUB - 1) * SUB, SUB)],
        out_hbm.at[pl.ds(base + (NSUB - 1) * SUB, SUB)], so))
    for co in cos:
        co.wait()


def kernel(input_ids, position_ids, wte, wpe):
    out = _embed(input_ids.astype(jnp.int32), position_ids.astype(jnp.int32),
                 wte, wpe)
    return out.reshape(input_ids.shape + (wte.shape[1],))


# speedup vs baseline: 1.0225x; 1.0055x over previous
import functools

import jax
import jax.numpy as jnp
from jax import lax
from jax.experimental import pallas as pl
from jax.experimental.pallas import tpu as pltpu
from jax.experimental.pallas import tpu_sc as plsc

D = 128
BATCH = 4
SEQ = 2048
N_TOK = BATCH * SEQ
NC = 2
NS = 16
NW = NC * NS
PER_W = N_TOK // NW
W_PER_ROW = SEQ // PER_W
SUB = 256
NSUB = PER_W // SUB

_mesh = plsc.VectorSubcoreMesh(core_axis_name="c", subcore_axis_name="s")


@functools.partial(
    pl.kernel,
    out_type=jax.ShapeDtypeStruct((N_TOK, D), jnp.float32),
    mesh=_mesh,
    scratch_types=[
        pltpu.VMEM((PER_W,), jnp.int32),
        pltpu.VMEM((PER_W,), jnp.int32),
        pltpu.VMEM((PER_W, D), jnp.float32),
        pltpu.SemaphoreType.DMA,
        pltpu.SemaphoreType.DMA,
        pltpu.SemaphoreType.DMA,
        pltpu.SemaphoreType.DMA,
        pltpu.SemaphoreType.DMA,
        pltpu.SemaphoreType.DMA,
        pltpu.SemaphoreType.DMA,
        pltpu.SemaphoreType.DMA,
        pltpu.SemaphoreType.DMA,
        pltpu.SemaphoreType.DMA,
        pltpu.SemaphoreType.DMA,
    ],
)
def _embed(ids_hbm, pos_hbm, wte_hbm, wpe_hbm, out_hbm,
           ti_v, pi_v, a,
           si0, si1, sa0, sa1, sa2, sa3, sb0, sb1, sb2, sb3, so):
    wid = lax.axis_index("s") * NC + lax.axis_index("c")
    brow = wid // W_PER_ROW
    s0 = (wid % W_PER_ROW) * PER_W
    ci0 = pltpu.async_copy(ids_hbm.at[brow, pl.ds(s0, PER_W)], ti_v, si0)
    ci1 = pltpu.async_copy(pos_hbm.at[brow, pl.ds(s0, PER_W)], pi_v, si1)
    sas = (sa0, sa1, sa2, sa3)
    sbs = (sb0, sb1, sb2, sb3)

    def _gather_wte(q):
        return pltpu.async_copy(
            wte_hbm.at[ti_v.at[pl.ds(q * SUB, SUB)]],
            a.at[pl.ds(q * SUB, SUB)], sas[q])

    ci0.wait()
    gas = [_gather_wte(q) for q in range(min(2, NSUB))]
    ci1.wait()
    base = wid * PER_W
    gbs, cos = [], []
    for q in range(NSUB):
        gas[q].wait()
        gbs.append(pltpu.async_copy(
            wpe_hbm.at[pi_v.at[pl.ds(q * SUB, SUB)]],
            a.at[pl.ds(q * SUB, SUB)], sbs[q], add=True))
        if q + 2 < NSUB:
            gas.append(_gather_wte(q + 2))
        if q >= 1:
            gbs[q - 1].wait()
            cos.append(pltpu.async_copy(
                a.at[pl.ds((q - 1) * SUB, SUB)],
                out_hbm.at[pl.ds(base + (q - 1) * SUB, SUB)], so))
    gbs[NSUB - 1].wait()
    cos.append(pltpu.async_copy(
        a.at[pl.ds((NSUB - 1) * SUB, SUB)],
        out_hbm.at[pl.ds(base + (NSUB - 1) * SUB, SUB)], so))
    for co in cos:
        co.wait()


def kernel(input_ids, position_ids, wte, wpe):
    out = _embed(input_ids.astype(jnp.int32), position_ids.astype(jnp.int32),
                 wte, wpe)
    return out.reshape(input_ids.shape + (wte.shape[1],))


# speedup vs baseline: 1.0282x; 1.0056x over previous
import functools

import jax
import jax.numpy as jnp
from jax import lax
from jax.experimental import pallas as pl
from jax.experimental.pallas import tpu as pltpu
from jax.experimental.pallas import tpu_sc as plsc

D = 128
BATCH = 4
SEQ = 2048
N_TOK = BATCH * SEQ
NC = 2
NS = 16
NW = NC * NS
PER_W = N_TOK // NW
W_PER_ROW = SEQ // PER_W

_mesh = plsc.VectorSubcoreMesh(core_axis_name="c", subcore_axis_name="s")


@functools.partial(
    pl.kernel,
    out_type=jax.ShapeDtypeStruct((N_TOK, D), jnp.float32),
    mesh=_mesh,
    scratch_types=[
        pltpu.VMEM((PER_W,), jnp.int32),
        pltpu.VMEM((PER_W,), jnp.int32),
        pltpu.VMEM((PER_W, D), jnp.float32),
        pltpu.SemaphoreType.DMA,
        pltpu.SemaphoreType.DMA,
        pltpu.SemaphoreType.DMA,
    ],
)
def _embed(ids_hbm, pos_hbm, wte_hbm, wpe_hbm, out_hbm,
           ti_v, pi_v, a, si0, si1, sg):
    wid = lax.axis_index("s") * NC + lax.axis_index("c")
    brow = wid // W_PER_ROW
    s0 = (wid % W_PER_ROW) * PER_W
    ci0 = pltpu.async_copy(ids_hbm.at[brow, pl.ds(s0, PER_W)], ti_v, si0)
    ci1 = pltpu.async_copy(pos_hbm.at[brow, pl.ds(s0, PER_W)], pi_v, si1)
    ci0.wait()
    ga = pltpu.async_copy(wte_hbm.at[ti_v], a, sg)
    ci1.wait()
    ga.wait()
    gb = pltpu.async_copy(wpe_hbm.at[pi_v], a, sg, add=True)
    gb.wait()
    co = pltpu.async_copy(a, out_hbm.at[pl.ds(wid * PER_W, PER_W)], sg)
    co.wait()


def kernel(input_ids, position_ids, wte, wpe):
    out = _embed(input_ids.astype(jnp.int32), position_ids.astype(jnp.int32),
                 wte, wpe)
    return out.reshape(input_ids.shape + (wte.shape[1],))
